# Initial kernel scaffold; baseline (speedup 1.0000x reference)
import functools
import jax
import jax.numpy as jnp
from jax import lax
from jax.experimental import pallas as pl
from jax.experimental.pallas import tpu as pltpu
from jax.experimental.pallas import tpu_sc as plsc

N = 10000
D = 128
H = 8
C = 128
HC = H * C

NC = 2
NS = 16
NW = NC * NS
L = 16

KB = 32
ROWS_PER_TILE = N // NS


def _leaky(a):
    return jnp.where(a >= 0.0, a, 0.2 * a)


_PR = 1000


def _proj_body(x_ref, w_ref, asrc_ref, adst_ref, h_ref, ta_ref, td_ref):
    x = x_ref[...]
    h = jnp.dot(x, w_ref[...], preferred_element_type=jnp.float32)
    h_ref[...] = h
    h3 = h.reshape(_PR, H, C)
    a_s = jnp.sum(h3 * asrc_ref[...][None], axis=-1)
    a_d = jnp.sum(h3 * adst_ref[...][None], axis=-1)
    ta_ref[...] = jnp.concatenate([a_s, a_s], axis=1)
    td_ref[...] = jnp.concatenate([a_d, a_d], axis=1)


def _project(x, W, a_src, a_dst):
    n = x.shape[0]
    grid = n // _PR
    return pl.pallas_call(
        _proj_body,
        grid=(grid,),
        in_specs=[
            pl.BlockSpec((_PR, D), lambda i: (i, 0)),
            pl.BlockSpec((D, HC), lambda i: (0, 0)),
            pl.BlockSpec((H, C), lambda i: (0, 0)),
            pl.BlockSpec((H, C), lambda i: (0, 0)),
        ],
        out_specs=[
            pl.BlockSpec((_PR, HC), lambda i: (i, 0)),
            pl.BlockSpec((_PR, 2 * H), lambda i: (i, 0)),
            pl.BlockSpec((_PR, 2 * H), lambda i: (i, 0)),
        ],
        out_shape=[
            jax.ShapeDtypeStruct((n, HC), jnp.float32),
            jax.ShapeDtypeStruct((n, 2 * H), jnp.float32),
            jax.ShapeDtypeStruct((n, 2 * H), jnp.float32),
        ],
    )(x, W, a_src, a_dst)


def _rcomb_body(d_ref, r_ref):
    d = d_ref[...]
    r_ref[...] = (1.0 / H) / (d[0] + d[1] + 1e-16)


def _rcomb(dparts):
    return pl.pallas_call(
        _rcomb_body,
        out_shape=jax.ShapeDtypeStruct((N, 2 * H), jnp.float32),
    )(dparts)


def _finalize_body(p_ref, x_ref, b_ref, w_ref, bb_ref, y_ref):
    p = p_ref[...]
    hh = x_ref[...] + p[0] + p[1] + b_ref[...]
    m = jnp.mean(hh)
    msq = jnp.mean(hh * hh)
    var = msq - m * m
    yn = (hh - m) * lax.rsqrt(var + 1e-5)
    y_ref[...] = jnp.maximum(yn * w_ref[...] + bb_ref[...], 0.0)


def _finalize(parts, xres, bias, lnw, lnb):
    return pl.pallas_call(
        _finalize_body,
        out_shape=jax.ShapeDtypeStruct((N, C), jnp.float32),
    )(parts, xres, bias.reshape(1, C), lnw.reshape(1, C), lnb.reshape(1, C))


def _bcast_lane(v, h):
    idx = jnp.full((L, 1), h, dtype=jnp.int32)
    dn = lax.GatherDimensionNumbers(
        offset_dims=(), collapsed_slice_dims=(0,), start_index_map=(0,))
    return lax.gather(v, idx, dn, (1,),
                      mode=lax.GatherScatterMode.PROMISE_IN_BOUNDS)


def _edge_mask(base_e, e, e_tot):
    eid = jnp.full((L,), base_e + e, dtype=jnp.int32)
    return jnp.where(eid < e_tot, 1.0, 0.0)


def _make_pass_a(e_pad, e_tot):
    epw = e_pad // NW
    nblk = epw // KB
    mesh = plsc.VectorSubcoreMesh(core_axis_name="c", subcore_axis_name="s")

    @functools.partial(
        pl.kernel,
        out_type=jax.ShapeDtypeStruct((NC, N, 2 * H), jnp.float32),
        mesh=mesh,
        scratch_types=[
            pltpu.VMEM((KB,), jnp.int32),
            pltpu.VMEM((KB,), jnp.int32),
            pltpu.VMEM((KB, 2 * H), jnp.float32),
            pltpu.VMEM((KB, 2 * H), jnp.float32),
            pltpu.VMEM((KB, 2 * H), jnp.float32),
            pltpu.VMEM((ROWS_PER_TILE, 2 * H), jnp.float32),
            pltpu.VMEM_SHARED((N, 2 * H), jnp.float32),
            pltpu.SemaphoreType.DMA,
            pltpu.SemaphoreType.DMA,
        ],
    )
    def pass_a(src_hbm, dst_hbm, ta_hbm, td_hbm, dpart_hbm,
               sidx, didx, tas, tdd, exb, zbuf, dacc, sem0, sem1):
        cid = lax.axis_index("c")
        sid = lax.axis_index("s")
        wid = sid * NC + cid
        wbase = wid * epw

        def zrow(i, _):
            zbuf[i, :] = jnp.zeros((L,), jnp.float32)
            return 0
        lax.fori_loop(0, ROWS_PER_TILE, zrow, 0)
        pltpu.sync_copy(zbuf, dacc.at[pl.ds(sid * ROWS_PER_TILE,
                                            ROWS_PER_TILE)])
        plsc.subcore_barrier()

        def blk(i, _):
            base_e = wbase + i * KB
            pltpu.sync_copy(src_hbm.at[pl.ds(base_e, KB)], sidx)
            pltpu.sync_copy(dst_hbm.at[pl.ds(base_e, KB)], didx)
            ca = pltpu.async_copy(ta_hbm.at[sidx], tas, sem0)
            cb = pltpu.async_copy(td_hbm.at[didx], tdd, sem1)
            ca.wait()
            cb.wait()
            for e in range(KB):
                a = tas[e, :] + tdd[e, :]
                ex = jnp.exp(_leaky(a)) * _edge_mask(base_e, e, e_tot)
                exb[e, :] = ex
            pltpu.sync_copy(exb, dacc.at[didx], add=True)
            return 0
        lax.fori_loop(0, nblk, blk, 0)

        plsc.subcore_barrier()
        pltpu.sync_copy(
            dacc.at[pl.ds(sid * ROWS_PER_TILE, ROWS_PER_TILE)],
            dpart_hbm.at[cid, pl.ds(sid * ROWS_PER_TILE, ROWS_PER_TILE)])

    return pass_a


def _make_pass_b(e_pad, e_tot):
    epw = e_pad // NW
    nblk = epw // KB
    mesh = plsc.VectorSubcoreMesh(core_axis_name="c", subcore_axis_name="s")
    zr = 125

    @functools.partial(
        pl.kernel,
        out_type=jax.ShapeDtypeStruct((NC, N, C), jnp.float32),
        mesh=mesh,
        scratch_types=[
            pltpu.VMEM((KB,), jnp.int32),
            pltpu.VMEM((KB,), jnp.int32),
            pltpu.VMEM((KB, 2 * H), jnp.float32),
            pltpu.VMEM((KB, 2 * H), jnp.float32),
            pltpu.VMEM((KB, 2 * H), jnp.float32),
            pltpu.VMEM((KB, HC), jnp.float32),
            pltpu.VMEM((KB, C), jnp.float32),
            pltpu.VMEM((125, C), jnp.float32),
            pltpu.VMEM_SHARED((N, C), jnp.float32),
            pltpu.SemaphoreType.DMA,
            pltpu.SemaphoreType.DMA,
            pltpu.SemaphoreType.DMA,
            pltpu.SemaphoreType.DMA,
        ],
    )
    def pass_b(src_hbm, dst_hbm, ta_hbm, td_hbm, r_hbm, h_hbm, opart_hbm,
               sidx, didx, tas, tdd, rd, hbuf, mbuf, zbuf, oacc,
               sem0, sem1, sem2, sem3):
        cid = lax.axis_index("c")
        sid = lax.axis_index("s")
        wid = sid * NC + cid
        wbase = wid * epw

        def zrow(i, _):
            for c in range(C // L):
                zbuf[i, pl.ds(c * L, L)] = jnp.zeros((L,), jnp.float32)
            return 0
        lax.fori_loop(0, zr, zrow, 0)
        for piece in range(ROWS_PER_TILE // zr):
            pltpu.sync_copy(
                zbuf, oacc.at[pl.ds(sid * ROWS_PER_TILE + piece * zr, zr)])
        plsc.subcore_barrier()

        def blk(i, _):
            base_e = wbase + i * KB
            pltpu.sync_copy(src_hbm.at[pl.ds(base_e, KB)], sidx)
            pltpu.sync_copy(dst_hbm.at[pl.ds(base_e, KB)], didx)
            c0 = pltpu.async_copy(ta_hbm.at[sidx], tas, sem0)
            c1 = pltpu.async_copy(td_hbm.at[didx], tdd, sem1)
            c2 = pltpu.async_copy(r_hbm.at[didx], rd, sem2)
            c3 = pltpu.async_copy(h_hbm.at[sidx], hbuf, sem3)
            c0.wait()
            c1.wait()
            c2.wait()
            c3.wait()
            for e in range(KB):
                a = tas[e, :] + tdd[e, :]
                att = (jnp.exp(_leaky(a)) * rd[e, :]
                       * _edge_mask(base_e, e, e_tot))
                for h in range(H):
                    ab = _bcast_lane(att, h)
                    for c in range(C // L):
                        piece = ab * hbuf[e, pl.ds(h * C + c * L, L)]
                        if h == 0:
                            mbuf[e, pl.ds(c * L, L)] = piece
                        else:
                            mbuf[e, pl.ds(c * L, L)] += piece
            pltpu.sync_copy(mbuf, oacc.at[didx], add=True)
            return 0
        lax.fori_loop(0, nblk, blk, 0)

        plsc.subcore_barrier()
        pltpu.sync_copy(
            oacc.at[pl.ds(sid * ROWS_PER_TILE, ROWS_PER_TILE)],
            opart_hbm.at[cid, pl.ds(sid * ROWS_PER_TILE, ROWS_PER_TILE)])

    return pass_b


def kernel(x, edge_index, batch, W1, a_src1, a_dst1, b1, lnw1, lnb1,
           W2, a_src2, a_dst2, b2, lnw2, lnb2):
    n = x.shape[0]
    e_in = edge_index.shape[1]
    e_tot = e_in + n
    epw = ((e_tot + NW * KB - 1) // (NW * KB)) * KB
    e_pad = epw * NW

    loop = jnp.arange(n, dtype=jnp.int32)
    pad = jnp.zeros((e_pad - e_tot,), dtype=jnp.int32)
    src = jnp.concatenate([edge_index[0], loop, pad])
    dst = jnp.concatenate([edge_index[1], loop, pad])

    pass_a = _make_pass_a(e_pad, e_tot)
    pass_b = _make_pass_b(e_pad, e_tot)

    def layer(xin, W, a_src, a_dst, bias, lnw, lnb):
        h, ta, td = _project(xin, W, a_src, a_dst)
        dparts = pass_a(src, dst, ta, td)
        r = _rcomb(dparts)
        oparts = pass_b(src, dst, ta, td, r, h)
        return _finalize(oparts, xin, bias, lnw, lnb)

    y1 = layer(x, W1, a_src1, a_dst1, b1, lnw1, lnb1)
    y2 = layer(y1, W2, a_src2, a_dst2, b2, lnw2, lnb2)
    return y2


# speedup vs baseline: 20.3900x; 20.3900x over previous
import functools
import jax
import jax.numpy as jnp
from jax import lax
from jax.experimental import pallas as pl
from jax.experimental.pallas import tpu as pltpu
from jax.experimental.pallas import tpu_sc as plsc

N = 10000
D = 128
H = 8
C = 128
HC = H * C

NC = 2
NS = 16
NW = NC * NS
L = 16

KB = 32
RB = 624
RT = N - NS * RB


def _leaky(a):
    return jnp.where(a >= 0.0, a, 0.2 * a)


_PR = 1000


def _proj_body(x_ref, w_ref, asrc_ref, adst_ref, h_ref, ta_ref, td_ref):
    x = x_ref[...]
    h = jnp.dot(x, w_ref[...], preferred_element_type=jnp.float32)
    h_ref[...] = h
    h3 = h.reshape(_PR, H, C)
    a_s = jnp.sum(h3 * asrc_ref[...][None], axis=-1)
    a_d = jnp.sum(h3 * adst_ref[...][None], axis=-1)
    ta_ref[...] = jnp.concatenate([a_s, a_s], axis=1)
    td_ref[...] = jnp.concatenate([a_d, a_d], axis=1)


def _project(x, W, a_src, a_dst):
    n = x.shape[0]
    grid = n // _PR
    return pl.pallas_call(
        _proj_body,
        grid=(grid,),
        in_specs=[
            pl.BlockSpec((_PR, D), lambda i: (i, 0)),
            pl.BlockSpec((D, HC), lambda i: (0, 0)),
            pl.BlockSpec((H, C), lambda i: (0, 0)),
            pl.BlockSpec((H, C), lambda i: (0, 0)),
        ],
        out_specs=[
            pl.BlockSpec((_PR, HC), lambda i: (i, 0)),
            pl.BlockSpec((_PR, 2 * H), lambda i: (i, 0)),
            pl.BlockSpec((_PR, 2 * H), lambda i: (i, 0)),
        ],
        out_shape=[
            jax.ShapeDtypeStruct((n, HC), jnp.float32),
            jax.ShapeDtypeStruct((n, 2 * H), jnp.float32),
            jax.ShapeDtypeStruct((n, 2 * H), jnp.float32),
        ],
    )(x, W, a_src, a_dst)


def _rcomb_body(d_ref, r_ref):
    d = d_ref[...]
    r_ref[...] = (1.0 / H) / (d[0] + d[1] + 1e-16)


def _rcomb(dparts):
    return pl.pallas_call(
        _rcomb_body,
        out_shape=jax.ShapeDtypeStruct((N, 2 * H), jnp.float32),
    )(dparts)


def _finalize_body(p_ref, x_ref, b_ref, w_ref, bb_ref, y_ref):
    p = p_ref[...]
    hh = x_ref[...] + p[0] + p[1] + b_ref[...]
    m = jnp.mean(hh)
    msq = jnp.mean(hh * hh)
    var = msq - m * m
    yn = (hh - m) * lax.rsqrt(var + 1e-5)
    y_ref[...] = jnp.maximum(yn * w_ref[...] + bb_ref[...], 0.0)


def _finalize(parts, xres, bias, lnw, lnb):
    return pl.pallas_call(
        _finalize_body,
        out_shape=jax.ShapeDtypeStruct((N, C), jnp.float32),
    )(parts, xres, bias.reshape(1, C), lnw.reshape(1, C), lnb.reshape(1, C))


def _bcast_lane(v, h):
    idx = jnp.full((L, 1), h, dtype=jnp.int32)
    dn = lax.GatherDimensionNumbers(
        offset_dims=(), collapsed_slice_dims=(0,), start_index_map=(0,))
    return lax.gather(v, idx, dn, (1,),
                      mode=lax.GatherScatterMode.PROMISE_IN_BOUNDS)


def _edge_mask(base_e, e, e_tot):
    eid = jnp.full((L,), base_e + e, dtype=jnp.int32)
    return jnp.where(eid < e_tot, 1.0, 0.0)


def _make_pass_a(e_pad, e_tot):
    epw = e_pad // NW
    nblk = epw // KB
    mesh = plsc.VectorSubcoreMesh(core_axis_name="c", subcore_axis_name="s")

    @functools.partial(
        pl.kernel,
        out_type=jax.ShapeDtypeStruct((NC, N, 2 * H), jnp.float32),
        mesh=mesh,
        compiler_params=pltpu.CompilerParams(use_tc_tiling_on_sc=False),
        scratch_types=[
            pltpu.VMEM((KB,), jnp.int32),
            pltpu.VMEM((KB,), jnp.int32),
            pltpu.VMEM((KB, 2 * H), jnp.float32),
            pltpu.VMEM((KB, 2 * H), jnp.float32),
            pltpu.VMEM((KB, 2 * H), jnp.float32),
            pltpu.VMEM((RB, 2 * H), jnp.float32),
            pltpu.VMEM_SHARED((N, 2 * H), jnp.float32),
            pltpu.SemaphoreType.DMA,
            pltpu.SemaphoreType.DMA,
        ],
    )
    def pass_a(src_hbm, dst_hbm, ta_hbm, td_hbm, dpart_hbm,
               sidx, didx, tas, tdd, exb, zbuf, dacc, sem0, sem1):
        cid = lax.axis_index("c")
        sid = lax.axis_index("s")
        wid = sid * NC + cid
        wbase = wid * epw

        def zrow(i, _):
            zbuf[i, :] = jnp.zeros((L,), jnp.float32)
            return 0
        lax.fori_loop(0, RB, zrow, 0)
        pltpu.sync_copy(zbuf, dacc.at[pl.ds(sid * RB, RB)])

        @pl.when(sid == 0)
        def _():
            pltpu.sync_copy(zbuf.at[pl.ds(0, RT)],
                            dacc.at[pl.ds(NS * RB, RT)])
        plsc.subcore_barrier()

        def blk(i, _):
            base_e = wbase + i * KB
            pltpu.sync_copy(src_hbm.at[pl.ds(base_e, KB)], sidx)
            pltpu.sync_copy(dst_hbm.at[pl.ds(base_e, KB)], didx)
            ca = pltpu.async_copy(ta_hbm.at[sidx], tas, sem0)
            cb = pltpu.async_copy(td_hbm.at[didx], tdd, sem1)
            ca.wait()
            cb.wait()
            for e in range(KB):
                a = tas[e, :] + tdd[e, :]
                ex = jnp.exp(_leaky(a)) * _edge_mask(base_e, e, e_tot)
                exb[e, :] = ex
            pltpu.sync_copy(exb, dacc.at[didx], add=True)
            return 0
        lax.fori_loop(0, nblk, blk, 0)

        plsc.subcore_barrier()
        pltpu.sync_copy(dacc.at[pl.ds(sid * RB, RB)],
                        dpart_hbm.at[cid, pl.ds(sid * RB, RB)])

        @pl.when(sid == 0)
        def _():
            pltpu.sync_copy(dacc.at[pl.ds(NS * RB, RT)],
                            dpart_hbm.at[cid, pl.ds(NS * RB, RT)])

    return pass_a


def _make_pass_b(e_pad, e_tot):
    epw = e_pad // NW
    nblk = epw // KB
    mesh = plsc.VectorSubcoreMesh(core_axis_name="c", subcore_axis_name="s")
    zr = 16

    @functools.partial(
        pl.kernel,
        out_type=jax.ShapeDtypeStruct((NC, N, C), jnp.float32),
        mesh=mesh,
        compiler_params=pltpu.CompilerParams(use_tc_tiling_on_sc=False),
        scratch_types=[
            pltpu.VMEM((KB,), jnp.int32),
            pltpu.VMEM((KB,), jnp.int32),
            pltpu.VMEM((KB, 2 * H), jnp.float32),
            pltpu.VMEM((KB, 2 * H), jnp.float32),
            pltpu.VMEM((KB, 2 * H), jnp.float32),
            pltpu.VMEM((KB, HC), jnp.float32),
            pltpu.VMEM((KB, C), jnp.float32),
            pltpu.VMEM((zr, C), jnp.float32),
            pltpu.VMEM_SHARED((N, C), jnp.float32),
            pltpu.SemaphoreType.DMA,
            pltpu.SemaphoreType.DMA,
            pltpu.SemaphoreType.DMA,
            pltpu.SemaphoreType.DMA,
        ],
    )
    def pass_b(src_hbm, dst_hbm, ta_hbm, td_hbm, r_hbm, h_hbm, opart_hbm,
               sidx, didx, tas, tdd, rd, hbuf, mbuf, zbuf, oacc,
               sem0, sem1, sem2, sem3):
        cid = lax.axis_index("c")
        sid = lax.axis_index("s")
        wid = sid * NC + cid
        wbase = wid * epw

        def zrow(i, _):
            for c in range(C // L):
                zbuf[i, pl.ds(c * L, L)] = jnp.zeros((L,), jnp.float32)
            return 0
        lax.fori_loop(0, zr, zrow, 0)
        for piece in range(RB // zr):
            pltpu.sync_copy(zbuf, oacc.at[pl.ds(sid * RB + piece * zr, zr)])

        @pl.when(sid == 0)
        def _():
            pltpu.sync_copy(zbuf.at[pl.ds(0, RT)],
                            oacc.at[pl.ds(NS * RB, RT)])
        plsc.subcore_barrier()

        def blk(i, _):
            base_e = wbase + i * KB
            pltpu.sync_copy(src_hbm.at[pl.ds(base_e, KB)], sidx)
            pltpu.sync_copy(dst_hbm.at[pl.ds(base_e, KB)], didx)
            c0 = pltpu.async_copy(ta_hbm.at[sidx], tas, sem0)
            c1 = pltpu.async_copy(td_hbm.at[didx], tdd, sem1)
            c2 = pltpu.async_copy(r_hbm.at[didx], rd, sem2)
            c3 = pltpu.async_copy(h_hbm.at[sidx], hbuf, sem3)
            c0.wait()
            c1.wait()
            c2.wait()
            c3.wait()

            def edge(e, _):
                a = tas[e, :] + tdd[e, :]
                att = (jnp.exp(_leaky(a)) * rd[e, :]
                       * _edge_mask(base_e, e, e_tot))
                m = [None] * (C // L)
                for h in range(H):
                    ab = _bcast_lane(att, h)
                    for c in range(C // L):
                        piece = ab * hbuf[e, pl.ds(h * C + c * L, L)]
                        m[c] = piece if h == 0 else m[c] + piece
                for c in range(C // L):
                    mbuf[e, pl.ds(c * L, L)] = m[c]
                return 0
            lax.fori_loop(0, KB, edge, 0)
            pltpu.sync_copy(mbuf, oacc.at[didx], add=True)
            return 0
        lax.fori_loop(0, nblk, blk, 0)

        plsc.subcore_barrier()
        pltpu.sync_copy(oacc.at[pl.ds(sid * RB, RB)],
                        opart_hbm.at[cid, pl.ds(sid * RB, RB)])

        @pl.when(sid == 0)
        def _():
            pltpu.sync_copy(oacc.at[pl.ds(NS * RB, RT)],
                            opart_hbm.at[cid, pl.ds(NS * RB, RT)])

    return pass_b


def kernel(x, edge_index, batch, W1, a_src1, a_dst1, b1, lnw1, lnb1,
           W2, a_src2, a_dst2, b2, lnw2, lnb2):
    n = x.shape[0]
    e_in = edge_index.shape[1]
    e_tot = e_in + n
    epw = ((e_tot + NW * KB - 1) // (NW * KB)) * KB
    e_pad = epw * NW

    loop = jnp.arange(n, dtype=jnp.int32)
    pad = jnp.zeros((e_pad - e_tot,), dtype=jnp.int32)
    src = jnp.concatenate([edge_index[0], loop, pad])
    dst = jnp.concatenate([edge_index[1], loop, pad])

    pass_a = _make_pass_a(e_pad, e_tot)
    pass_b = _make_pass_b(e_pad, e_tot)

    def layer(xin, W, a_src, a_dst, bias, lnw, lnb):
        h, ta, td = _project(xin, W, a_src, a_dst)
        dparts = pass_a(src, dst, ta, td)
        r = _rcomb(dparts)
        oparts = pass_b(src, dst, ta, td, r, h)
        return _finalize(oparts, xin, bias, lnw, lnb)

    y1 = layer(x, W1, a_src1, a_dst1, b1, lnw1, lnb1)
    y2 = layer(y1, W2, a_src2, a_dst2, b2, lnw2, lnb2)
    return y2


# speedup vs baseline: 40.0396x; 1.9637x over previous
import functools
import jax
import jax.numpy as jnp
from jax import lax
from jax.experimental import pallas as pl
from jax.experimental.pallas import tpu as pltpu
from jax.experimental.pallas import tpu_sc as plsc

N = 10000
D = 128
H = 8
C = 128
HC = H * C

NC = 2
NS = 16
NW = NC * NS
L = 16

KA = 128
KB = 16
RB = 624
RT = N - NS * RB


def _leaky(a):
    return jnp.where(a >= 0.0, a, 0.2 * a)


_PR = 1000


def _proj_body(x_ref, w_ref, asrc_ref, adst_ref, h_ref, ta_ref, td_ref):
    x = x_ref[...]
    h = jnp.dot(x, w_ref[...], preferred_element_type=jnp.float32)
    h_ref[...] = h
    h3 = h.reshape(_PR, H, C)
    a_s = jnp.sum(h3 * asrc_ref[...][None], axis=-1)
    a_d = jnp.sum(h3 * adst_ref[...][None], axis=-1)
    ta_ref[...] = jnp.concatenate([a_s, a_s], axis=1)
    td_ref[...] = jnp.concatenate([a_d, a_d], axis=1)


def _project(x, W, a_src, a_dst):
    n = x.shape[0]
    grid = n // _PR
    return pl.pallas_call(
        _proj_body,
        grid=(grid,),
        in_specs=[
            pl.BlockSpec((_PR, D), lambda i: (i, 0)),
            pl.BlockSpec((D, HC), lambda i: (0, 0)),
            pl.BlockSpec((H, C), lambda i: (0, 0)),
            pl.BlockSpec((H, C), lambda i: (0, 0)),
        ],
        out_specs=[
            pl.BlockSpec((_PR, HC), lambda i: (i, 0)),
            pl.BlockSpec((_PR, 2 * H), lambda i: (i, 0)),
            pl.BlockSpec((_PR, 2 * H), lambda i: (i, 0)),
        ],
        out_shape=[
            jax.ShapeDtypeStruct((n, HC), jnp.float32),
            jax.ShapeDtypeStruct((n, 2 * H), jnp.float32),
            jax.ShapeDtypeStruct((n, 2 * H), jnp.float32),
        ],
    )(x, W, a_src, a_dst)


def _rcomb_body(d_ref, td_ref, t_ref):
    d = d_ref[...]
    r = (1.0 / H) / (d[0, :, :H] + d[1, :, :H] + 1e-16)
    t_ref[...] = jnp.concatenate([td_ref[...][:, :H], r], axis=1)


def _rcomb(dparts, td):
    return pl.pallas_call(
        _rcomb_body,
        out_shape=jax.ShapeDtypeStruct((N, 2 * H), jnp.float32),
    )(dparts, td)


def _finalize_body(p_ref, x_ref, b_ref, w_ref, bb_ref, y_ref):
    p = p_ref[...]
    hh = x_ref[...] + p[0] + p[1] + b_ref[...]
    m = jnp.mean(hh)
    msq = jnp.mean(hh * hh)
    var = msq - m * m
    yn = (hh - m) * lax.rsqrt(var + 1e-5)
    y_ref[...] = jnp.maximum(yn * w_ref[...] + bb_ref[...], 0.0)


def _finalize(parts, xres, bias, lnw, lnb):
    return pl.pallas_call(
        _finalize_body,
        out_shape=jax.ShapeDtypeStruct((N, C), jnp.float32),
    )(parts, xres, bias.reshape(1, C), lnw.reshape(1, C), lnb.reshape(1, C))


_GDN = lax.GatherDimensionNumbers(
    offset_dims=(), collapsed_slice_dims=(0,), start_index_map=(0,))


def _bcast_lane(v, h):
    idx = jnp.full((L, 1), h, dtype=jnp.int32)
    return lax.gather(v, idx, _GDN, (1,),
                      mode=lax.GatherScatterMode.PROMISE_IN_BOUNDS)


def _upper_half(v):
    idx = (jnp.arange(L, dtype=jnp.int32) % 8 + 8)[:, None]
    return lax.gather(v, idx, _GDN, (1,),
                      mode=lax.GatherScatterMode.PROMISE_IN_BOUNDS)


def _edge_mask(base_e, e, e_tot):
    eid = jnp.full((L,), base_e + e, dtype=jnp.int32)
    return jnp.where(eid < e_tot, 1.0, 0.0)


def _zero_rows(zbuf, nrows):
    def zrow(i, _):
        for c in range(zbuf.shape[1] // L):
            zbuf[i, pl.ds(c * L, L)] = jnp.zeros((L,), jnp.float32)
        return 0
    lax.fori_loop(0, nrows, zrow, 0)


def _init_acc(zbuf, acc, sid):
    zr = zbuf.shape[0]
    _zero_rows(zbuf, zr)
    for piece in range(RB // zr):
        pltpu.sync_copy(zbuf, acc.at[pl.ds(sid * RB + piece * zr, zr)])

    @pl.when(sid == 0)
    def _():
        pltpu.sync_copy(zbuf.at[pl.ds(0, RT)], acc.at[pl.ds(NS * RB, RT)])
    plsc.subcore_barrier()


def _writeback(acc, out_hbm, cid, sid):
    plsc.subcore_barrier()
    pltpu.sync_copy(acc.at[pl.ds(sid * RB, RB)],
                    out_hbm.at[cid, pl.ds(sid * RB, RB)])

    @pl.when(sid == 0)
    def _():
        pltpu.sync_copy(acc.at[pl.ds(NS * RB, RT)],
                        out_hbm.at[cid, pl.ds(NS * RB, RT)])


def _snapshot_idx(dst_ref, src_ref, k):
    def cp(j, _):
        dst_ref[pl.ds(j * L, L)] = src_ref[pl.ds(j * L, L)]
        return 0
    lax.fori_loop(0, k // L, cp, 0)


def _make_pass_a(e_pad, e_tot):
    epw = e_pad // NW
    nblk = epw // KA
    nk = nblk // 2
    mesh = plsc.VectorSubcoreMesh(core_axis_name="c", subcore_axis_name="s")

    slot_scratch = [
        pltpu.VMEM((KA,), jnp.int32),
        pltpu.VMEM((KA,), jnp.int32),
        pltpu.VMEM((KA,), jnp.int32),
        pltpu.VMEM((KA, 2 * H), jnp.float32),
        pltpu.VMEM((KA, 2 * H), jnp.float32),
        pltpu.VMEM((KA, 2 * H), jnp.float32),
        pltpu.SemaphoreType.DMA,
        pltpu.SemaphoreType.DMA,
        pltpu.SemaphoreType.DMA,
    ]

    @functools.partial(
        pl.kernel,
        out_type=jax.ShapeDtypeStruct((NC, N, 2 * H), jnp.float32),
        mesh=mesh,
        compiler_params=pltpu.CompilerParams(use_tc_tiling_on_sc=False),
        scratch_types=slot_scratch + slot_scratch + [
            pltpu.VMEM((RB, 2 * H), jnp.float32),
            pltpu.VMEM_SHARED((N, 2 * H), jnp.float32),
        ],
    )
    def pass_a(src_hbm, dst_hbm, ta_hbm, td_hbm, dpart_hbm, *bufs):
        slots = (bufs[0:9], bufs[9:18])
        zbuf, dacc = bufs[18], bufs[19]
        cid = lax.axis_index("c")
        sid = lax.axis_index("s")
        wid = sid * NC + cid
        wbase = wid * epw

        _init_acc(zbuf, dacc, sid)

        def idx_copies(i, b):
            s = slots[b]
            base_e = wbase + i * KA
            return (pltpu.make_async_copy(
                        src_hbm.at[pl.ds(base_e, KA)], s[0], s[6]),
                    pltpu.make_async_copy(
                        dst_hbm.at[pl.ds(base_e, KA)], s[1], s[6]))

        def gather_copies(b):
            s = slots[b]
            return (pltpu.make_async_copy(ta_hbm.at[s[0]], s[3], s[7]),
                    pltpu.make_async_copy(td_hbm.at[s[1]], s[4], s[7]))

        def scatter_copy(b):
            s = slots[b]
            return pltpu.make_async_copy(s[5], dacc.at[s[2]], s[8])

        def issue_idx(i, b):
            for cpy in idx_copies(i, b):
                cpy.start()

        def wait_idx_issue_gathers(i, b):
            for cpy in idx_copies(i, b):
                cpy.wait()
            for cpy in gather_copies(b):
                cpy.start()

        def do_block(i, b, first, last):
            s = slots[b]
            base_e = wbase + i * KA
            for cpy in gather_copies(b):
                cpy.wait()

            @pl.when(jnp.logical_not(first))
            def _():
                scatter_copy(b).wait()
            _snapshot_idx(s[2], s[1], KA)

            @pl.when(jnp.logical_not(last))
            def _():
                issue_idx(i + 2, b)

            def edge(e, _):
                a = s[3][e, :] + s[4][e, :]
                ex = jnp.exp(_leaky(a)) * _edge_mask(base_e, e, e_tot)
                s[5][e, :] = ex
                return 0
            lax.fori_loop(0, KA, edge, 0)
            scatter_copy(b).start(add=True)

            @pl.when(jnp.logical_not(last))
            def _():
                wait_idx_issue_gathers(i + 2, b)

        issue_idx(0, 0)
        issue_idx(1, 1)
        wait_idx_issue_gathers(0, 0)
        wait_idx_issue_gathers(1, 1)

        def body(k, _):
            first = k == 0
            last = k == nk - 1
            do_block(2 * k, 0, first, last)
            do_block(2 * k + 1, 1, first, last)
            return 0
        lax.fori_loop(0, nk, body, 0)

        scatter_copy(0).wait()
        scatter_copy(1).wait()
        _writeback(dacc, dpart_hbm, cid, sid)

    return pass_a


def _make_pass_b(e_pad, e_tot):
    epw = e_pad // NW
    nblk = epw // KB
    nk = nblk // 2
    mesh = plsc.VectorSubcoreMesh(core_axis_name="c", subcore_axis_name="s")

    slot_scratch = [
        pltpu.VMEM((KB,), jnp.int32),
        pltpu.VMEM((KB,), jnp.int32),
        pltpu.VMEM((KB,), jnp.int32),
        pltpu.VMEM((KB, 2 * H), jnp.float32),
        pltpu.VMEM((KB, 2 * H), jnp.float32),
        pltpu.VMEM((KB, HC), jnp.float32),
        pltpu.VMEM((KB, C), jnp.float32),
        pltpu.SemaphoreType.DMA,
        pltpu.SemaphoreType.DMA,
        pltpu.SemaphoreType.DMA,
    ]

    @functools.partial(
        pl.kernel,
        out_type=jax.ShapeDtypeStruct((NC, N, C), jnp.float32),
        mesh=mesh,
        compiler_params=pltpu.CompilerParams(use_tc_tiling_on_sc=False),
        scratch_types=slot_scratch + slot_scratch + [
            pltpu.VMEM((16, C), jnp.float32),
            pltpu.VMEM_SHARED((N, C), jnp.float32),
        ],
    )
    def pass_b(src_hbm, dst_hbm, ta_hbm, trd_hbm, h_hbm, opart_hbm, *bufs):
        slots = (bufs[0:10], bufs[10:20])
        zbuf, oacc = bufs[20], bufs[21]
        cid = lax.axis_index("c")
        sid = lax.axis_index("s")
        wid = sid * NC + cid
        wbase = wid * epw

        _init_acc(zbuf, oacc, sid)

        def idx_copies(i, b):
            s = slots[b]
            base_e = wbase + i * KB
            return (pltpu.make_async_copy(
                        src_hbm.at[pl.ds(base_e, KB)], s[0], s[7]),
                    pltpu.make_async_copy(
                        dst_hbm.at[pl.ds(base_e, KB)], s[1], s[7]))

        def gather_copies(b):
            s = slots[b]
            return (pltpu.make_async_copy(ta_hbm.at[s[0]], s[3], s[8]),
                    pltpu.make_async_copy(trd_hbm.at[s[1]], s[4], s[8]),
                    pltpu.make_async_copy(h_hbm.at[s[0]], s[5], s[8]))

        def scatter_copy(b):
            s = slots[b]
            return pltpu.make_async_copy(s[6], oacc.at[s[2]], s[9])

        def issue_idx(i, b):
            for cpy in idx_copies(i, b):
                cpy.start()

        def wait_idx_issue_gathers(i, b):
            for cpy in idx_copies(i, b):
                cpy.wait()
            for cpy in gather_copies(b):
                cpy.start()

        def do_block(i, b, first, last):
            s = slots[b]
            base_e = wbase + i * KB
            for cpy in gather_copies(b):
                cpy.wait()

            @pl.when(jnp.logical_not(first))
            def _():
                scatter_copy(b).wait()
            _snapshot_idx(s[2], s[1], KB)

            @pl.when(jnp.logical_not(last))
            def _():
                issue_idx(i + 2, b)

            tas, trd, hbuf, mbuf = s[3], s[4], s[5], s[6]

            def edge(e, _):
                a = tas[e, :] + trd[e, :]
                ex = jnp.exp(_leaky(a))
                r_al = _upper_half(trd[e, :])
                att = ex * r_al * _edge_mask(base_e, e, e_tot)
                m = [None] * (C // L)
                for h in range(H):
                    ab = _bcast_lane(att, h)
                    for c in range(C // L):
                        piece = ab * hbuf[e, pl.ds(h * C + c * L, L)]
                        m[c] = piece if h == 0 else m[c] + piece
                for c in range(C // L):
                    mbuf[e, pl.ds(c * L, L)] = m[c]
                return 0
            lax.fori_loop(0, KB, edge, 0)
            scatter_copy(b).start(add=True)

            @pl.when(jnp.logical_not(last))
            def _():
                wait_idx_issue_gathers(i + 2, b)

        issue_idx(0, 0)
        issue_idx(1, 1)
        wait_idx_issue_gathers(0, 0)
        wait_idx_issue_gathers(1, 1)

        def body(k, _):
            first = k == 0
            last = k == nk - 1
            do_block(2 * k, 0, first, last)
            do_block(2 * k + 1, 1, first, last)
            return 0
        lax.fori_loop(0, nk, body, 0)

        scatter_copy(0).wait()
        scatter_copy(1).wait()
        _writeback(oacc, opart_hbm, cid, sid)

    return pass_b


def kernel(x, edge_index, batch, W1, a_src1, a_dst1, b1, lnw1, lnb1,
           W2, a_src2, a_dst2, b2, lnw2, lnb2):
    n = x.shape[0]
    e_in = edge_index.shape[1]
    e_tot = e_in + n
    q = 2 * KA
    epw = ((e_tot + NW * q - 1) // (NW * q)) * q
    e_pad = epw * NW

    loop = jnp.arange(n, dtype=jnp.int32)
    pad = jnp.zeros((e_pad - e_tot,), dtype=jnp.int32)
    src = jnp.concatenate([edge_index[0], loop, pad])
    dst = jnp.concatenate([edge_index[1], loop, pad])

    pass_a = _make_pass_a(e_pad, e_tot)
    pass_b = _make_pass_b(e_pad, e_tot)

    def layer(xin, W, a_src, a_dst, bias, lnw, lnb):
        h, ta, td = _project(xin, W, a_src, a_dst)
        dparts = pass_a(src, dst, ta, td)
        trd = _rcomb(dparts, td)
        oparts = pass_b(src, dst, ta, trd, h)
        return _finalize(oparts, xin, bias, lnw, lnb)

    y1 = layer(x, W1, a_src1, a_dst1, b1, lnw1, lnb1)
    y2 = layer(y1, W2, a_src2, a_dst2, b2, lnw2, lnb2)
    return y2


# speedup vs baseline: 44.7279x; 1.1171x over previous
import functools
import jax
import numpy as np
import jax.numpy as jnp
from jax import lax
from jax.experimental import pallas as pl
from jax.experimental.pallas import tpu as pltpu
from jax.experimental.pallas import tpu_sc as plsc

N = 10000
D = 128
H = 8
C = 128
HC = H * C

NC = 2
NS = 16
NW = NC * NS
L = 16

KA = 128
KB = 16
RB = 624
RT = N - NS * RB


def _leaky(a):
    return jnp.where(a >= 0.0, a, 0.2 * a)


_PR = 2000


def _proj_body(x_ref, w_ref, asrc_ref, adst_ref, h_ref, ta_ref, td_ref):
    x = x_ref[...]
    h = jnp.dot(x, w_ref[...], preferred_element_type=jnp.float32)
    h_ref[...] = h.astype(jnp.bfloat16)
    h3 = h.reshape(_PR, H, C)
    a_s = jnp.sum(h3 * asrc_ref[...][None], axis=-1)
    a_d = jnp.sum(h3 * adst_ref[...][None], axis=-1)
    ta_ref[...] = jnp.concatenate([a_s, a_s], axis=1)
    td_ref[...] = jnp.concatenate([a_d, a_d], axis=1)


def _project(x, W, a_src, a_dst):
    n = x.shape[0]
    grid = n // _PR
    return pl.pallas_call(
        _proj_body,
        grid=(grid,),
        in_specs=[
            pl.BlockSpec((_PR, D), lambda i: (i, 0)),
            pl.BlockSpec((D, HC), lambda i: (0, 0)),
            pl.BlockSpec((H, C), lambda i: (0, 0)),
            pl.BlockSpec((H, C), lambda i: (0, 0)),
        ],
        out_specs=[
            pl.BlockSpec((_PR, HC), lambda i: (i, 0)),
            pl.BlockSpec((_PR, 2 * H), lambda i: (i, 0)),
            pl.BlockSpec((_PR, 2 * H), lambda i: (i, 0)),
        ],
        out_shape=[
            jax.ShapeDtypeStruct((n, HC), jnp.bfloat16),
            jax.ShapeDtypeStruct((n, 2 * H), jnp.float32),
            jax.ShapeDtypeStruct((n, 2 * H), jnp.float32),
        ],
    )(x, W, a_src, a_dst)


def _rcomb_body(d_ref, td_ref, t_ref):
    d = d_ref[...]
    r = (1.0 / H) / (d[0, :, :H] + d[1, :, :H] + 1e-16)
    t_ref[...] = jnp.concatenate([td_ref[...][:, :H], r], axis=1)


def _rcomb(dparts, td):
    return pl.pallas_call(
        _rcomb_body,
        out_shape=jax.ShapeDtypeStruct((N, 2 * H), jnp.float32),
    )(dparts, td)


def _unshuffle_perm():
    p = np.zeros((C, C), np.float32)
    for j in range(C):
        q, r = divmod(j, 32)
        k, par = divmod(r, 2)
        p[32 * q + 16 * par + k, j] = 1.0
    return p


_UNSHUF = _unshuffle_perm()


def _finalize_body(p_ref, x_ref, perm_ref, b_ref, w_ref, bb_ref, y_ref):
    p = p_ref[...]
    o = jnp.dot(p[0] + p[1], perm_ref[...],
                preferred_element_type=jnp.float32)
    hh = x_ref[...] + o + b_ref[...]
    m = jnp.mean(hh)
    msq = jnp.mean(hh * hh)
    var = msq - m * m
    yn = (hh - m) * lax.rsqrt(var + 1e-5)
    y_ref[...] = jnp.maximum(yn * w_ref[...] + bb_ref[...], 0.0)


def _finalize(parts, xres, bias, lnw, lnb):
    return pl.pallas_call(
        _finalize_body,
        out_shape=jax.ShapeDtypeStruct((N, C), jnp.float32),
    )(parts, xres, jnp.asarray(_UNSHUF), bias.reshape(1, C),
      lnw.reshape(1, C), lnb.reshape(1, C))


_GDN = lax.GatherDimensionNumbers(
    offset_dims=(), collapsed_slice_dims=(0,), start_index_map=(0,))


def _bcast_lane(v, h):
    idx = jnp.full((L, 1), h, dtype=jnp.int32)
    return lax.gather(v, idx, _GDN, (1,),
                      mode=lax.GatherScatterMode.PROMISE_IN_BOUNDS)


def _upper_half(v):
    idx = (jnp.arange(L, dtype=jnp.int32) % 8 + 8)[:, None]
    return lax.gather(v, idx, _GDN, (1,),
                      mode=lax.GatherScatterMode.PROMISE_IN_BOUNDS)


def _edge_mask(base_e, e, e_tot):
    eid = jnp.full((L,), base_e + e, dtype=jnp.int32)
    return jnp.where(eid < e_tot, 1.0, 0.0)


def _zero_rows(zbuf, nrows):
    def zrow(i, _):
        for c in range(zbuf.shape[1] // L):
            zbuf[i, pl.ds(c * L, L)] = jnp.zeros((L,), jnp.float32)
        return 0
    lax.fori_loop(0, nrows, zrow, 0)


def _init_acc(zbuf, acc, sid):
    zr = zbuf.shape[0]
    _zero_rows(zbuf, zr)
    for piece in range(RB // zr):
        pltpu.sync_copy(zbuf, acc.at[pl.ds(sid * RB + piece * zr, zr)])

    @pl.when(sid == 0)
    def _():
        pltpu.sync_copy(zbuf.at[pl.ds(0, RT)], acc.at[pl.ds(NS * RB, RT)])
    plsc.subcore_barrier()


def _writeback(acc, out_hbm, cid, sid):
    plsc.subcore_barrier()
    pltpu.sync_copy(acc.at[pl.ds(sid * RB, RB)],
                    out_hbm.at[cid, pl.ds(sid * RB, RB)])

    @pl.when(sid == 0)
    def _():
        pltpu.sync_copy(acc.at[pl.ds(NS * RB, RT)],
                        out_hbm.at[cid, pl.ds(NS * RB, RT)])


def _snapshot_idx(dst_ref, src_ref, k):
    def cp(j, _):
        dst_ref[pl.ds(j * L, L)] = src_ref[pl.ds(j * L, L)]
        return 0
    lax.fori_loop(0, k // L, cp, 0)


def _make_pass_a(e_pad, e_tot):
    epw = e_pad // NW
    nblk = epw // KA
    nk = nblk // 2
    mesh = plsc.VectorSubcoreMesh(core_axis_name="c", subcore_axis_name="s")

    slot_scratch = [
        pltpu.VMEM((KA,), jnp.int32),
        pltpu.VMEM((KA,), jnp.int32),
        pltpu.VMEM((KA,), jnp.int32),
        pltpu.VMEM((KA, 2 * H), jnp.float32),
        pltpu.VMEM((KA, 2 * H), jnp.float32),
        pltpu.VMEM((KA, 2 * H), jnp.float32),
        pltpu.SemaphoreType.DMA,
        pltpu.SemaphoreType.DMA,
        pltpu.SemaphoreType.DMA,
    ]

    @functools.partial(
        pl.kernel,
        out_type=jax.ShapeDtypeStruct((NC, N, 2 * H), jnp.float32),
        mesh=mesh,
        compiler_params=pltpu.CompilerParams(
            use_tc_tiling_on_sc=False, needs_layout_passes=False),
        scratch_types=slot_scratch + slot_scratch + [
            pltpu.VMEM((RB, 2 * H), jnp.float32),
            pltpu.VMEM_SHARED((N, 2 * H), jnp.float32),
        ],
    )
    def pass_a(src_hbm, dst_hbm, ta_hbm, td_hbm, dpart_hbm, *bufs):
        slots = (bufs[0:9], bufs[9:18])
        zbuf, dacc = bufs[18], bufs[19]
        cid = lax.axis_index("c")
        sid = lax.axis_index("s")
        wid = sid * NC + cid
        wbase = wid * epw

        _init_acc(zbuf, dacc, sid)

        def idx_copies(i, b):
            s = slots[b]
            base_e = wbase + i * KA
            return (pltpu.make_async_copy(
                        src_hbm.at[pl.ds(base_e, KA)], s[0], s[6]),
                    pltpu.make_async_copy(
                        dst_hbm.at[pl.ds(base_e, KA)], s[1], s[6]))

        def gather_copies(b):
            s = slots[b]
            return (pltpu.make_async_copy(ta_hbm.at[s[0]], s[3], s[7]),
                    pltpu.make_async_copy(td_hbm.at[s[1]], s[4], s[7]))

        def scatter_copy(b):
            s = slots[b]
            return pltpu.make_async_copy(s[5], dacc.at[s[2]], s[8])

        def issue_idx(i, b):
            for cpy in idx_copies(i, b):
                cpy.start()

        def wait_idx_issue_gathers(i, b):
            for cpy in idx_copies(i, b):
                cpy.wait()
            for cpy in gather_copies(b):
                cpy.start()

        def do_block(i, b, first, last):
            s = slots[b]
            base_e = wbase + i * KA
            for cpy in gather_copies(b):
                cpy.wait()

            @pl.when(jnp.logical_not(first))
            def _():
                scatter_copy(b).wait()
            _snapshot_idx(s[2], s[1], KA)

            @pl.when(jnp.logical_not(last))
            def _():
                issue_idx(i + 2, b)

            def edge(e, _):
                a = s[3][e, :] + s[4][e, :]
                ex = jnp.exp(_leaky(a)) * _edge_mask(base_e, e, e_tot)
                s[5][e, :] = ex
                return 0
            lax.fori_loop(0, KA, edge, 0)
            scatter_copy(b).start(add=True)

            @pl.when(jnp.logical_not(last))
            def _():
                wait_idx_issue_gathers(i + 2, b)

        issue_idx(0, 0)
        issue_idx(1, 1)
        wait_idx_issue_gathers(0, 0)
        wait_idx_issue_gathers(1, 1)

        def body(k, _):
            first = k == 0
            last = k == nk - 1
            do_block(2 * k, 0, first, last)
            do_block(2 * k + 1, 1, first, last)
            return 0
        lax.fori_loop(0, nk, body, 0)

        scatter_copy(0).wait()
        scatter_copy(1).wait()
        _writeback(dacc, dpart_hbm, cid, sid)

    return pass_a


def _make_pass_b(e_pad, e_tot):
    epw = e_pad // NW
    nblk = epw // KB
    nk = nblk // 2
    mesh = plsc.VectorSubcoreMesh(core_axis_name="c", subcore_axis_name="s")

    slot_scratch = [
        pltpu.VMEM((KB,), jnp.int32),
        pltpu.VMEM((KB,), jnp.int32),
        pltpu.VMEM((KB,), jnp.int32),
        pltpu.VMEM((KB, 2 * H), jnp.float32),
        pltpu.VMEM((KB, 2 * H), jnp.float32),
        pltpu.VMEM((KB, HC), jnp.bfloat16),
        pltpu.VMEM((KB, C), jnp.float32),
        pltpu.SemaphoreType.DMA,
        pltpu.SemaphoreType.DMA,
        pltpu.SemaphoreType.DMA,
    ]

    @functools.partial(
        pl.kernel,
        out_type=jax.ShapeDtypeStruct((NC, N, C), jnp.float32),
        mesh=mesh,
        compiler_params=pltpu.CompilerParams(
            use_tc_tiling_on_sc=False, needs_layout_passes=False),
        scratch_types=slot_scratch + slot_scratch + [
            pltpu.VMEM((16, C), jnp.float32),
            pltpu.VMEM_SHARED((N, C), jnp.float32),
        ],
    )
    def pass_b(src_hbm, dst_hbm, ta_hbm, trd_hbm, h_hbm, opart_hbm, *bufs):
        slots = (bufs[0:10], bufs[10:20])
        zbuf, oacc = bufs[20], bufs[21]
        cid = lax.axis_index("c")
        sid = lax.axis_index("s")
        wid = sid * NC + cid
        wbase = wid * epw

        _init_acc(zbuf, oacc, sid)

        def idx_copies(i, b):
            s = slots[b]
            base_e = wbase + i * KB
            return (pltpu.make_async_copy(
                        src_hbm.at[pl.ds(base_e, KB)], s[0], s[7]),
                    pltpu.make_async_copy(
                        dst_hbm.at[pl.ds(base_e, KB)], s[1], s[7]))

        def gather_copies(b):
            s = slots[b]
            return (pltpu.make_async_copy(ta_hbm.at[s[0]], s[3], s[8]),
                    pltpu.make_async_copy(trd_hbm.at[s[1]], s[4], s[8]),
                    pltpu.make_async_copy(h_hbm.at[s[0]], s[5], s[8]))

        def scatter_copy(b):
            s = slots[b]
            return pltpu.make_async_copy(s[6], oacc.at[s[2]], s[9])

        def issue_idx(i, b):
            for cpy in idx_copies(i, b):
                cpy.start()

        def wait_idx_issue_gathers(i, b):
            for cpy in idx_copies(i, b):
                cpy.wait()
            for cpy in gather_copies(b):
                cpy.start()

        def do_block(i, b, first, last):
            s = slots[b]
            base_e = wbase + i * KB
            for cpy in gather_copies(b):
                cpy.wait()

            @pl.when(jnp.logical_not(first))
            def _():
                scatter_copy(b).wait()
            _snapshot_idx(s[2], s[1], KB)

            @pl.when(jnp.logical_not(last))
            def _():
                issue_idx(i + 2, b)

            tas, trd, hbuf, mbuf = s[3], s[4], s[5], s[6]

            def edge(e, _):
                a = tas[e, :] + trd[e, :]
                ex = jnp.exp(_leaky(a))
                r_al = _upper_half(trd[e, :])
                att = ex * r_al * _edge_mask(base_e, e, e_tot)
                mev = [None] * 4
                mod = [None] * 4
                for h in range(H):
                    ab = _bcast_lane(att, h)
                    for g in range(4):
                        v = hbuf[e, pl.ds(h * C + g * 32, 32)]
                        pa, pb = plsc.unpack(
                            v, format=plsc.PackFormat.INTERLEAVED)
                        if h == 0:
                            mev[g] = ab * pa
                            mod[g] = ab * pb
                        else:
                            mev[g] += ab * pa
                            mod[g] += ab * pb
                for g in range(4):
                    mbuf[e, pl.ds(g * 32, L)] = mev[g]
                    mbuf[e, pl.ds(g * 32 + L, L)] = mod[g]
                return 0
            lax.fori_loop(0, KB, edge, 0)
            scatter_copy(b).start(add=True)

            @pl.when(jnp.logical_not(last))
            def _():
                wait_idx_issue_gathers(i + 2, b)

        issue_idx(0, 0)
        issue_idx(1, 1)
        wait_idx_issue_gathers(0, 0)
        wait_idx_issue_gathers(1, 1)

        def body(k, _):
            first = k == 0
            last = k == nk - 1
            do_block(2 * k, 0, first, last)
            do_block(2 * k + 1, 1, first, last)
            return 0
        lax.fori_loop(0, nk, body, 0)

        scatter_copy(0).wait()
        scatter_copy(1).wait()
        _writeback(oacc, opart_hbm, cid, sid)

    return pass_b


def kernel(x, edge_index, batch, W1, a_src1, a_dst1, b1, lnw1, lnb1,
           W2, a_src2, a_dst2, b2, lnw2, lnb2):
    n = x.shape[0]
    e_in = edge_index.shape[1]
    e_tot = e_in + n
    q = 2 * KA
    epw = ((e_tot + NW * q - 1) // (NW * q)) * q
    e_pad = epw * NW

    loop = jnp.arange(n, dtype=jnp.int32)
    pad = jnp.zeros((e_pad - e_tot,), dtype=jnp.int32)
    src = jnp.concatenate([edge_index[0], loop, pad])
    dst = jnp.concatenate([edge_index[1], loop, pad])

    pass_a = _make_pass_a(e_pad, e_tot)
    pass_b = _make_pass_b(e_pad, e_tot)

    def layer(xin, W, a_src, a_dst, bias, lnw, lnb):
        h, ta, td = _project(xin, W, a_src, a_dst)
        dparts = pass_a(src, dst, ta, td)
        trd = _rcomb(dparts, td)
        oparts = pass_b(src, dst, ta, trd, h)
        return _finalize(oparts, xin, bias, lnw, lnb)

    y1 = layer(x, W1, a_src1, a_dst1, b1, lnw1, lnb1)
    y2 = layer(y1, W2, a_src2, a_dst2, b2, lnw2, lnb2)
    return y2


# speedup vs baseline: 44.7519x; 1.0005x over previous
import functools
import jax
import numpy as np
import jax.numpy as jnp
from jax import lax
from jax.experimental import pallas as pl
from jax.experimental.pallas import tpu as pltpu
from jax.experimental.pallas import tpu_sc as plsc

N = 10000
D = 128
H = 8
C = 128
HC = H * C

NC = 2
NS = 16
NW = NC * NS
L = 16

KA = 128
KB = 16
RB = 624
RT = N - NS * RB


def _leaky(a):
    return jnp.where(a >= 0.0, a, 0.2 * a)


_PR = 2000


def _proj_body(x_ref, w_ref, asrc_ref, adst_ref, h_ref, ta_ref, td_ref):
    x = x_ref[...]
    h = jnp.dot(x, w_ref[...], preferred_element_type=jnp.float32)
    h_ref[...] = h.astype(jnp.bfloat16)
    h3 = h.reshape(_PR, H, C)
    a_s = jnp.sum(h3 * asrc_ref[...][None], axis=-1)
    a_d = jnp.sum(h3 * adst_ref[...][None], axis=-1)
    ta_ref[...] = jnp.concatenate([a_s, a_s], axis=1)
    td_ref[...] = jnp.concatenate([a_d, a_d], axis=1)


def _project(x, W, a_src, a_dst):
    n = x.shape[0]
    grid = n // _PR
    return pl.pallas_call(
        _proj_body,
        grid=(grid,),
        in_specs=[
            pl.BlockSpec((_PR, D), lambda i: (i, 0)),
            pl.BlockSpec((D, HC), lambda i: (0, 0)),
            pl.BlockSpec((H, C), lambda i: (0, 0)),
            pl.BlockSpec((H, C), lambda i: (0, 0)),
        ],
        out_specs=[
            pl.BlockSpec((_PR, HC), lambda i: (i, 0)),
            pl.BlockSpec((_PR, 2 * H), lambda i: (i, 0)),
            pl.BlockSpec((_PR, 2 * H), lambda i: (i, 0)),
        ],
        out_shape=[
            jax.ShapeDtypeStruct((n, HC), jnp.bfloat16),
            jax.ShapeDtypeStruct((n, 2 * H), jnp.float32),
            jax.ShapeDtypeStruct((n, 2 * H), jnp.float32),
        ],
    )(x, W, a_src, a_dst)


def _rcomb_body(d_ref, td_ref, t_ref):
    d = d_ref[...]
    r = (1.0 / H) / (d[0, :, :H] + d[1, :, :H] + 1e-16)
    t_ref[...] = jnp.concatenate([td_ref[...][:, :H], r], axis=1)


def _rcomb(dparts, td):
    return pl.pallas_call(
        _rcomb_body,
        out_shape=jax.ShapeDtypeStruct((N, 2 * H), jnp.float32),
    )(dparts, td)


def _unshuffle_perm():
    p = np.zeros((C, C), np.float32)
    for j in range(C):
        q, r = divmod(j, 32)
        k, par = divmod(r, 2)
        p[32 * q + 16 * par + k, j] = 1.0
    return p


_UNSHUF = _unshuffle_perm()


def _finalize_body(p_ref, x_ref, perm_ref, b_ref, w_ref, bb_ref, y_ref):
    p = p_ref[...]
    o = jnp.dot(p[0] + p[1], perm_ref[...],
                preferred_element_type=jnp.float32)
    hh = x_ref[...] + o + b_ref[...]
    m = jnp.mean(hh)
    msq = jnp.mean(hh * hh)
    var = msq - m * m
    yn = (hh - m) * lax.rsqrt(var + 1e-5)
    y_ref[...] = jnp.maximum(yn * w_ref[...] + bb_ref[...], 0.0)


def _finalize(parts, xres, bias, lnw, lnb):
    return pl.pallas_call(
        _finalize_body,
        out_shape=jax.ShapeDtypeStruct((N, C), jnp.float32),
    )(parts, xres, jnp.asarray(_UNSHUF), bias.reshape(1, C),
      lnw.reshape(1, C), lnb.reshape(1, C))


_GDN = lax.GatherDimensionNumbers(
    offset_dims=(), collapsed_slice_dims=(0,), start_index_map=(0,))


def _bcast_lane(v, h):
    idx = jnp.full((L, 1), h, dtype=jnp.int32)
    return lax.gather(v, idx, _GDN, (1,),
                      mode=lax.GatherScatterMode.PROMISE_IN_BOUNDS)


def _upper_half(v):
    idx = (jnp.arange(L, dtype=jnp.int32) % 8 + 8)[:, None]
    return lax.gather(v, idx, _GDN, (1,),
                      mode=lax.GatherScatterMode.PROMISE_IN_BOUNDS)


def _edge_mask(base_e, e, e_tot):
    eid = jnp.full((L,), base_e + e, dtype=jnp.int32)
    return jnp.where(eid < e_tot, 1.0, 0.0)


def _zero_rows(zbuf, nrows):
    def zrow(i, _):
        for c in range(zbuf.shape[1] // L):
            zbuf[i, pl.ds(c * L, L)] = jnp.zeros((L,), jnp.float32)
        return 0
    lax.fori_loop(0, nrows, zrow, 0)


def _init_acc(zbuf, acc, sid):
    zr = zbuf.shape[0]
    _zero_rows(zbuf, zr)
    for piece in range(RB // zr):
        pltpu.sync_copy(zbuf, acc.at[pl.ds(sid * RB + piece * zr, zr)])

    @pl.when(sid == 0)
    def _():
        pltpu.sync_copy(zbuf.at[pl.ds(0, RT)], acc.at[pl.ds(NS * RB, RT)])
    plsc.subcore_barrier()


def _writeback(acc, out_hbm, cid, sid):
    plsc.subcore_barrier()
    pltpu.sync_copy(acc.at[pl.ds(sid * RB, RB)],
                    out_hbm.at[cid, pl.ds(sid * RB, RB)])

    @pl.when(sid == 0)
    def _():
        pltpu.sync_copy(acc.at[pl.ds(NS * RB, RT)],
                        out_hbm.at[cid, pl.ds(NS * RB, RT)])


def _snapshot_idx(dst_ref, src_ref, k):
    def cp(j, _):
        dst_ref[pl.ds(j * L, L)] = src_ref[pl.ds(j * L, L)]
        return 0
    lax.fori_loop(0, k // L, cp, 0)


def _make_pass_a(e_pad, e_tot):
    epw = e_pad // NW
    nblk = epw // KA
    nk = nblk // 2
    mesh = plsc.VectorSubcoreMesh(core_axis_name="c", subcore_axis_name="s")

    slot_scratch = [
        pltpu.VMEM((KA,), jnp.int32),
        pltpu.VMEM((KA,), jnp.int32),
        pltpu.VMEM((KA,), jnp.int32),
        pltpu.VMEM((KA, 2 * H), jnp.float32),
        pltpu.VMEM((KA, 2 * H), jnp.float32),
        pltpu.VMEM((KA, 2 * H), jnp.float32),
        pltpu.SemaphoreType.DMA,
        pltpu.SemaphoreType.DMA,
        pltpu.SemaphoreType.DMA,
    ]

    @functools.partial(
        pl.kernel,
        out_type=jax.ShapeDtypeStruct((NC, N, 2 * H), jnp.float32),
        mesh=mesh,
        compiler_params=pltpu.CompilerParams(
            use_tc_tiling_on_sc=False, needs_layout_passes=False),
        scratch_types=slot_scratch + slot_scratch + [
            pltpu.VMEM((RB, 2 * H), jnp.float32),
            pltpu.VMEM_SHARED((N, 2 * H), jnp.float32),
        ],
    )
    def pass_a(src_hbm, dst_hbm, ta_hbm, td_hbm, dpart_hbm, *bufs):
        slots = (bufs[0:9], bufs[9:18])
        zbuf, dacc = bufs[18], bufs[19]
        cid = lax.axis_index("c")
        sid = lax.axis_index("s")
        wid = sid * NC + cid
        wbase = wid * epw

        _init_acc(zbuf, dacc, sid)

        def idx_copies(i, b):
            s = slots[b]
            base_e = wbase + i * KA
            return (pltpu.make_async_copy(
                        src_hbm.at[pl.ds(base_e, KA)], s[0], s[6]),
                    pltpu.make_async_copy(
                        dst_hbm.at[pl.ds(base_e, KA)], s[1], s[6]))

        def gather_copies(b):
            s = slots[b]
            return (pltpu.make_async_copy(ta_hbm.at[s[0]], s[3], s[7]),
                    pltpu.make_async_copy(td_hbm.at[s[1]], s[4], s[7]))

        def scatter_copy(b):
            s = slots[b]
            return pltpu.make_async_copy(s[5], dacc.at[s[2]], s[8])

        def issue_idx(i, b):
            for cpy in idx_copies(i, b):
                cpy.start()

        def wait_idx_issue_gathers(i, b):
            for cpy in idx_copies(i, b):
                cpy.wait()
            for cpy in gather_copies(b):
                cpy.start()

        def do_block(i, b, first, last):
            s = slots[b]
            base_e = wbase + i * KA
            for cpy in gather_copies(b):
                cpy.wait()

            @pl.when(jnp.logical_not(first))
            def _():
                scatter_copy(b).wait()
            _snapshot_idx(s[2], s[1], KA)

            @pl.when(jnp.logical_not(last))
            def _():
                issue_idx(i + 2, b)

            def edge(e, _):
                a = s[3][e, :] + s[4][e, :]
                ex = jnp.exp(_leaky(a)) * _edge_mask(base_e, e, e_tot)
                s[5][e, :] = ex
                return 0
            lax.fori_loop(0, KA, edge, 0)
            scatter_copy(b).start(add=True)

            @pl.when(jnp.logical_not(last))
            def _():
                wait_idx_issue_gathers(i + 2, b)

        issue_idx(0, 0)
        issue_idx(1, 1)
        wait_idx_issue_gathers(0, 0)
        wait_idx_issue_gathers(1, 1)

        def body(k, _):
            first = k == 0
            last = k == nk - 1
            do_block(2 * k, 0, first, last)
            do_block(2 * k + 1, 1, first, last)
            return 0
        lax.fori_loop(0, nk, body, 0)

        scatter_copy(0).wait()
        scatter_copy(1).wait()
        _writeback(dacc, dpart_hbm, cid, sid)

    return pass_a


def _make_pass_b(e_pad, e_tot):
    epw = e_pad // NW
    nblk = epw // KB
    nk = nblk // 2
    mesh = plsc.VectorSubcoreMesh(core_axis_name="c", subcore_axis_name="s")

    slot_scratch = [
        pltpu.VMEM((KB,), jnp.int32),
        pltpu.VMEM((KB,), jnp.int32),
        pltpu.VMEM((KB,), jnp.int32),
        pltpu.VMEM((KB, 2 * H), jnp.float32),
        pltpu.VMEM((KB, 2 * H), jnp.float32),
        pltpu.VMEM((KB, HC), jnp.bfloat16),
        pltpu.VMEM((KB, C), jnp.float32),
        pltpu.SemaphoreType.DMA,
        pltpu.SemaphoreType.DMA,
        pltpu.SemaphoreType.DMA,
    ]

    @functools.partial(
        pl.kernel,
        out_type=jax.ShapeDtypeStruct((NC, N, C), jnp.float32),
        mesh=mesh,
        compiler_params=pltpu.CompilerParams(
            use_tc_tiling_on_sc=False, needs_layout_passes=False),
        scratch_types=slot_scratch + slot_scratch + [
            pltpu.VMEM((16, C), jnp.float32),
            pltpu.VMEM_SHARED((N, C), jnp.float32),
        ],
    )
    def pass_b(src_hbm, dst_hbm, ta_hbm, trd_hbm, h_hbm, opart_hbm, *bufs):
        slots = (bufs[0:10], bufs[10:20])
        zbuf, oacc = bufs[20], bufs[21]
        cid = lax.axis_index("c")
        sid = lax.axis_index("s")
        wid = sid * NC + cid
        wbase = wid * epw

        _init_acc(zbuf, oacc, sid)

        def idx_copies(i, b):
            s = slots[b]
            base_e = wbase + i * KB
            return (pltpu.make_async_copy(
                        src_hbm.at[pl.ds(base_e, KB)], s[0], s[7]),
                    pltpu.make_async_copy(
                        dst_hbm.at[pl.ds(base_e, KB)], s[1], s[7]))

        def gather_copies(b):
            s = slots[b]
            return (pltpu.make_async_copy(ta_hbm.at[s[0]], s[3], s[8]),
                    pltpu.make_async_copy(trd_hbm.at[s[1]], s[4], s[8]),
                    pltpu.make_async_copy(h_hbm.at[s[0]], s[5], s[8]))

        def scatter_copy(b):
            s = slots[b]
            return pltpu.make_async_copy(s[6], oacc.at[s[2]], s[9])

        def issue_idx(i, b):
            for cpy in idx_copies(i, b):
                cpy.start()

        def wait_idx_issue_gathers(i, b):
            for cpy in idx_copies(i, b):
                cpy.wait()
            for cpy in gather_copies(b):
                cpy.start()

        def do_block(i, b, first, last):
            s = slots[b]
            base_e = wbase + i * KB
            for cpy in gather_copies(b):
                cpy.wait()

            @pl.when(jnp.logical_not(first))
            def _():
                scatter_copy(b).wait()
            _snapshot_idx(s[2], s[1], KB)

            @pl.when(jnp.logical_not(last))
            def _():
                issue_idx(i + 2, b)

            tas, trd, hbuf, mbuf = s[3], s[4], s[5], s[6]

            def edge_pair(j, _):
                for u in range(2):
                    e = 2 * j + u
                    a = tas[e, :] + trd[e, :]
                    ex = jnp.exp(_leaky(a))
                    r_al = _upper_half(trd[e, :])
                    att = ex * r_al * _edge_mask(base_e, e, e_tot)
                    mev = [None] * 4
                    mod = [None] * 4
                    for h in range(H):
                        ab = _bcast_lane(att, h)
                        for g in range(4):
                            v = hbuf[e, pl.ds(h * C + g * 32, 32)]
                            pa, pb = plsc.unpack(
                                v, format=plsc.PackFormat.INTERLEAVED)
                            if h == 0:
                                mev[g] = ab * pa
                                mod[g] = ab * pb
                            else:
                                mev[g] += ab * pa
                                mod[g] += ab * pb
                    for g in range(4):
                        mbuf[e, pl.ds(g * 32, L)] = mev[g]
                        mbuf[e, pl.ds(g * 32 + L, L)] = mod[g]
                return 0
            lax.fori_loop(0, KB // 2, edge_pair, 0)
            scatter_copy(b).start(add=True)

            @pl.when(jnp.logical_not(last))
            def _():
                wait_idx_issue_gathers(i + 2, b)

        issue_idx(0, 0)
        issue_idx(1, 1)
        wait_idx_issue_gathers(0, 0)
        wait_idx_issue_gathers(1, 1)

        def body(k, _):
            first = k == 0
            last = k == nk - 1
            do_block(2 * k, 0, first, last)
            do_block(2 * k + 1, 1, first, last)
            return 0
        lax.fori_loop(0, nk, body, 0)

        scatter_copy(0).wait()
        scatter_copy(1).wait()
        _writeback(oacc, opart_hbm, cid, sid)

    return pass_b


def kernel(x, edge_index, batch, W1, a_src1, a_dst1, b1, lnw1, lnb1,
           W2, a_src2, a_dst2, b2, lnw2, lnb2):
    n = x.shape[0]
    e_in = edge_index.shape[1]
    e_tot = e_in + n
    q = 2 * KA
    epw = ((e_tot + NW * q - 1) // (NW * q)) * q
    e_pad = epw * NW

    loop = jnp.arange(n, dtype=jnp.int32)
    pad = jnp.zeros((e_pad - e_tot,), dtype=jnp.int32)
    src = jnp.concatenate([edge_index[0], loop, pad])
    dst = jnp.concatenate([edge_index[1], loop, pad])

    pass_a = _make_pass_a(e_pad, e_tot)
    pass_b = _make_pass_b(e_pad, e_tot)

    def layer(xin, W, a_src, a_dst, bias, lnw, lnb):
        h, ta, td = _project(xin, W, a_src, a_dst)
        dparts = pass_a(src, dst, ta, td)
        trd = _rcomb(dparts, td)
        oparts = pass_b(src, dst, ta, trd, h)
        return _finalize(oparts, xin, bias, lnw, lnb)

    y1 = layer(x, W1, a_src1, a_dst1, b1, lnw1, lnb1)
    y2 = layer(y1, W2, a_src2, a_dst2, b2, lnw2, lnb2)
    return y2


# speedup vs baseline: 45.2044x; 1.0101x over previous
import functools
import jax
import numpy as np
import jax.numpy as jnp
from jax import lax
from jax.experimental import pallas as pl
from jax.experimental.pallas import tpu as pltpu
from jax.experimental.pallas import tpu_sc as plsc

N = 10000
D = 128
H = 8
C = 128
HC = H * C

NC = 2
NS = 16
NW = NC * NS
L = 16

KA = 128
KB = 16
RB = 624
RT = N - NS * RB


def _leaky(a):
    return jnp.where(a >= 0.0, a, 0.2 * a)


_PR = 2000


def _proj_body(x_ref, w_ref, asrc_ref, adst_ref, h_ref, ta_ref, td_ref):
    x = x_ref[...]
    w = w_ref[...]
    h = jnp.dot(x.astype(jnp.bfloat16), w,
                preferred_element_type=jnp.float32)
    h_ref[...] = h.astype(jnp.bfloat16)
    wf = w.astype(jnp.float32)
    was = (wf * asrc_ref[...].reshape(1, HC)).reshape(D, H, C).sum(-1)
    wad = (wf * adst_ref[...].reshape(1, HC)).reshape(D, H, C).sum(-1)
    a_s = jnp.dot(x, was, preferred_element_type=jnp.float32)
    a_d = jnp.dot(x, wad, preferred_element_type=jnp.float32)
    ta_ref[...] = jnp.concatenate([a_s, a_s], axis=1)
    td_ref[...] = jnp.concatenate([a_d, a_d], axis=1)


def _project(x, W, a_src, a_dst):
    n = x.shape[0]
    grid = n // _PR
    return pl.pallas_call(
        _proj_body,
        grid=(grid,),
        in_specs=[
            pl.BlockSpec((_PR, D), lambda i: (i, 0)),
            pl.BlockSpec((D, HC), lambda i: (0, 0)),
            pl.BlockSpec((H, C), lambda i: (0, 0)),
            pl.BlockSpec((H, C), lambda i: (0, 0)),
        ],
        out_specs=[
            pl.BlockSpec((_PR, HC), lambda i: (i, 0)),
            pl.BlockSpec((_PR, 2 * H), lambda i: (i, 0)),
            pl.BlockSpec((_PR, 2 * H), lambda i: (i, 0)),
        ],
        out_shape=[
            jax.ShapeDtypeStruct((n, HC), jnp.bfloat16),
            jax.ShapeDtypeStruct((n, 2 * H), jnp.float32),
            jax.ShapeDtypeStruct((n, 2 * H), jnp.float32),
        ],
    )(x, W, a_src, a_dst)


def _rcomb_body(d_ref, td_ref, t_ref):
    d = d_ref[...]
    r = (1.0 / H) / (d[0, :, :H] + d[1, :, :H] + 1e-16)
    t_ref[...] = jnp.concatenate([td_ref[...][:, :H], r], axis=1)


def _rcomb(dparts, td):
    return pl.pallas_call(
        _rcomb_body,
        out_shape=jax.ShapeDtypeStruct((N, 2 * H), jnp.float32),
    )(dparts, td)


def _unshuffle_perm():
    p = np.zeros((C, C), np.float32)
    for j in range(C):
        q, r = divmod(j, 32)
        k, par = divmod(r, 2)
        p[32 * q + 16 * par + k, j] = 1.0
    return p


_UNSHUF = _unshuffle_perm()


def _finalize_body(p_ref, x_ref, perm_ref, b_ref, w_ref, bb_ref, y_ref):
    p = p_ref[...]
    o = jnp.dot(p[0] + p[1], perm_ref[...],
                preferred_element_type=jnp.float32)
    hh = x_ref[...] + o + b_ref[...]
    m = jnp.mean(hh)
    msq = jnp.mean(hh * hh)
    var = msq - m * m
    yn = (hh - m) * lax.rsqrt(var + 1e-5)
    y_ref[...] = jnp.maximum(yn * w_ref[...] + bb_ref[...], 0.0)


def _finalize(parts, xres, bias, lnw, lnb):
    return pl.pallas_call(
        _finalize_body,
        out_shape=jax.ShapeDtypeStruct((N, C), jnp.float32),
    )(parts, xres, jnp.asarray(_UNSHUF), bias.reshape(1, C),
      lnw.reshape(1, C), lnb.reshape(1, C))


_GDN = lax.GatherDimensionNumbers(
    offset_dims=(), collapsed_slice_dims=(0,), start_index_map=(0,))


def _bcast_lane(v, h):
    idx = jnp.full((L, 1), h, dtype=jnp.int32)
    return lax.gather(v, idx, _GDN, (1,),
                      mode=lax.GatherScatterMode.PROMISE_IN_BOUNDS)


def _upper_half(v):
    idx = (jnp.arange(L, dtype=jnp.int32) % 8 + 8)[:, None]
    return lax.gather(v, idx, _GDN, (1,),
                      mode=lax.GatherScatterMode.PROMISE_IN_BOUNDS)


def _edge_mask(base_e, e, e_tot):
    eid = jnp.full((L,), base_e + e, dtype=jnp.int32)
    return jnp.where(eid < e_tot, 1.0, 0.0)


def _zero_rows(zbuf, nrows):
    def zrow(i, _):
        for c in range(zbuf.shape[1] // L):
            zbuf[i, pl.ds(c * L, L)] = jnp.zeros((L,), jnp.float32)
        return 0
    lax.fori_loop(0, nrows, zrow, 0)


def _init_acc(zbuf, acc, sid):
    zr = zbuf.shape[0]
    _zero_rows(zbuf, zr)
    for piece in range(RB // zr):
        pltpu.sync_copy(zbuf, acc.at[pl.ds(sid * RB + piece * zr, zr)])

    @pl.when(sid == 0)
    def _():
        pltpu.sync_copy(zbuf.at[pl.ds(0, RT)], acc.at[pl.ds(NS * RB, RT)])
    plsc.subcore_barrier()


def _writeback(acc, out_hbm, cid, sid):
    plsc.subcore_barrier()
    pltpu.sync_copy(acc.at[pl.ds(sid * RB, RB)],
                    out_hbm.at[cid, pl.ds(sid * RB, RB)])

    @pl.when(sid == 0)
    def _():
        pltpu.sync_copy(acc.at[pl.ds(NS * RB, RT)],
                        out_hbm.at[cid, pl.ds(NS * RB, RT)])


def _snapshot_idx(dst_ref, src_ref, k):
    def cp(j, _):
        dst_ref[pl.ds(j * L, L)] = src_ref[pl.ds(j * L, L)]
        return 0
    lax.fori_loop(0, k // L, cp, 0)


def _make_pass_a(e_pad, e_tot):
    epw = e_pad // NW
    nblk = epw // KA
    nk = nblk // 2
    mesh = plsc.VectorSubcoreMesh(core_axis_name="c", subcore_axis_name="s")

    slot_scratch = [
        pltpu.VMEM((KA,), jnp.int32),
        pltpu.VMEM((KA,), jnp.int32),
        pltpu.VMEM((KA,), jnp.int32),
        pltpu.VMEM((KA, 2 * H), jnp.float32),
        pltpu.VMEM((KA, 2 * H), jnp.float32),
        pltpu.VMEM((KA, 2 * H), jnp.float32),
        pltpu.SemaphoreType.DMA,
        pltpu.SemaphoreType.DMA,
        pltpu.SemaphoreType.DMA,
    ]

    @functools.partial(
        pl.kernel,
        out_type=jax.ShapeDtypeStruct((NC, N, 2 * H), jnp.float32),
        mesh=mesh,
        compiler_params=pltpu.CompilerParams(
            use_tc_tiling_on_sc=False, needs_layout_passes=False),
        scratch_types=slot_scratch + slot_scratch + [
            pltpu.VMEM((RB, 2 * H), jnp.float32),
            pltpu.VMEM_SHARED((N, 2 * H), jnp.float32),
        ],
    )
    def pass_a(src_hbm, dst_hbm, ta_hbm, td_hbm, dpart_hbm, *bufs):
        slots = (bufs[0:9], bufs[9:18])
        zbuf, dacc = bufs[18], bufs[19]
        cid = lax.axis_index("c")
        sid = lax.axis_index("s")
        wid = sid * NC + cid
        wbase = wid * epw

        _init_acc(zbuf, dacc, sid)

        def idx_copies(i, b):
            s = slots[b]
            base_e = wbase + i * KA
            return (pltpu.make_async_copy(
                        src_hbm.at[pl.ds(base_e, KA)], s[0], s[6]),
                    pltpu.make_async_copy(
                        dst_hbm.at[pl.ds(base_e, KA)], s[1], s[6]))

        def gather_copies(b):
            s = slots[b]
            return (pltpu.make_async_copy(ta_hbm.at[s[0]], s[3], s[7]),
                    pltpu.make_async_copy(td_hbm.at[s[1]], s[4], s[7]))

        def scatter_copy(b):
            s = slots[b]
            return pltpu.make_async_copy(s[5], dacc.at[s[2]], s[8])

        def issue_idx(i, b):
            for cpy in idx_copies(i, b):
                cpy.start()

        def wait_idx_issue_gathers(i, b):
            for cpy in idx_copies(i, b):
                cpy.wait()
            for cpy in gather_copies(b):
                cpy.start()

        def do_block(i, b, first, last):
            s = slots[b]
            base_e = wbase + i * KA
            for cpy in gather_copies(b):
                cpy.wait()

            @pl.when(jnp.logical_not(first))
            def _():
                scatter_copy(b).wait()
            _snapshot_idx(s[2], s[1], KA)

            @pl.when(jnp.logical_not(last))
            def _():
                issue_idx(i + 2, b)

            def edge(e, _):
                a = s[3][e, :] + s[4][e, :]
                ex = jnp.exp(_leaky(a)) * _edge_mask(base_e, e, e_tot)
                s[5][e, :] = ex
                return 0
            lax.fori_loop(0, KA, edge, 0)
            scatter_copy(b).start(add=True)

            @pl.when(jnp.logical_not(last))
            def _():
                wait_idx_issue_gathers(i + 2, b)

        issue_idx(0, 0)
        issue_idx(1, 1)
        wait_idx_issue_gathers(0, 0)
        wait_idx_issue_gathers(1, 1)

        def body(k, _):
            first = k == 0
            last = k == nk - 1
            do_block(2 * k, 0, first, last)
            do_block(2 * k + 1, 1, first, last)
            return 0
        lax.fori_loop(0, nk, body, 0)

        scatter_copy(0).wait()
        scatter_copy(1).wait()
        _writeback(dacc, dpart_hbm, cid, sid)

    return pass_a


def _make_pass_b(e_pad, e_tot):
    epw = e_pad // NW
    nblk = epw // KB
    nk = nblk // 2
    mesh = plsc.VectorSubcoreMesh(core_axis_name="c", subcore_axis_name="s")

    slot_scratch = [
        pltpu.VMEM((KB,), jnp.int32),
        pltpu.VMEM((KB,), jnp.int32),
        pltpu.VMEM((KB,), jnp.int32),
        pltpu.VMEM((KB, 2 * H), jnp.float32),
        pltpu.VMEM((KB, 2 * H), jnp.float32),
        pltpu.VMEM((KB, HC), jnp.bfloat16),
        pltpu.VMEM((KB, C), jnp.float32),
        pltpu.SemaphoreType.DMA,
        pltpu.SemaphoreType.DMA,
        pltpu.SemaphoreType.DMA,
    ]

    @functools.partial(
        pl.kernel,
        out_type=jax.ShapeDtypeStruct((NC, N, C), jnp.float32),
        mesh=mesh,
        compiler_params=pltpu.CompilerParams(
            use_tc_tiling_on_sc=False, needs_layout_passes=False),
        scratch_types=slot_scratch + slot_scratch + [
            pltpu.VMEM((16, C), jnp.float32),
            pltpu.VMEM_SHARED((N, C), jnp.float32),
        ],
    )
    def pass_b(src_hbm, dst_hbm, ta_hbm, trd_hbm, h_hbm, opart_hbm, *bufs):
        slots = (bufs[0:10], bufs[10:20])
        zbuf, oacc = bufs[20], bufs[21]
        cid = lax.axis_index("c")
        sid = lax.axis_index("s")
        wid = sid * NC + cid
        wbase = wid * epw

        _init_acc(zbuf, oacc, sid)

        def idx_copies(i, b):
            s = slots[b]
            base_e = wbase + i * KB
            return (pltpu.make_async_copy(
                        src_hbm.at[pl.ds(base_e, KB)], s[0], s[7]),
                    pltpu.make_async_copy(
                        dst_hbm.at[pl.ds(base_e, KB)], s[1], s[7]))

        def gather_copies(b):
            s = slots[b]
            return (pltpu.make_async_copy(ta_hbm.at[s[0]], s[3], s[8]),
                    pltpu.make_async_copy(trd_hbm.at[s[1]], s[4], s[8]),
                    pltpu.make_async_copy(h_hbm.at[s[0]], s[5], s[8]))

        def scatter_copy(b):
            s = slots[b]
            return pltpu.make_async_copy(s[6], oacc.at[s[2]], s[9])

        def issue_idx(i, b):
            for cpy in idx_copies(i, b):
                cpy.start()

        def wait_idx_issue_gathers(i, b):
            for cpy in idx_copies(i, b):
                cpy.wait()
            for cpy in gather_copies(b):
                cpy.start()

        def do_block(i, b, first, last):
            s = slots[b]
            base_e = wbase + i * KB
            for cpy in gather_copies(b):
                cpy.wait()

            @pl.when(jnp.logical_not(first))
            def _():
                scatter_copy(b).wait()
            _snapshot_idx(s[2], s[1], KB)

            @pl.when(jnp.logical_not(last))
            def _():
                issue_idx(i + 2, b)

            tas, trd, hbuf, mbuf = s[3], s[4], s[5], s[6]

            def edge_pair(j, _):
                for u in range(2):
                    e = 2 * j + u
                    a = tas[e, :] + trd[e, :]
                    ex = jnp.exp(_leaky(a))
                    r_al = _upper_half(trd[e, :])
                    att = ex * r_al * _edge_mask(base_e, e, e_tot)
                    mev = [None] * 4
                    mod = [None] * 4
                    for h in range(H):
                        ab = _bcast_lane(att, h)
                        for g in range(4):
                            v = hbuf[e, pl.ds(h * C + g * 32, 32)]
                            pa, pb = plsc.unpack(
                                v, format=plsc.PackFormat.INTERLEAVED)
                            if h == 0:
                                mev[g] = ab * pa
                                mod[g] = ab * pb
                            else:
                                mev[g] += ab * pa
                                mod[g] += ab * pb
                    for g in range(4):
                        mbuf[e, pl.ds(g * 32, L)] = mev[g]
                        mbuf[e, pl.ds(g * 32 + L, L)] = mod[g]
                return 0
            lax.fori_loop(0, KB // 2, edge_pair, 0)
            scatter_copy(b).start(add=True)

            @pl.when(jnp.logical_not(last))
            def _():
                wait_idx_issue_gathers(i + 2, b)

        issue_idx(0, 0)
        issue_idx(1, 1)
        wait_idx_issue_gathers(0, 0)
        wait_idx_issue_gathers(1, 1)

        def body(k, _):
            first = k == 0
            last = k == nk - 1
            do_block(2 * k, 0, first, last)
            do_block(2 * k + 1, 1, first, last)
            return 0
        lax.fori_loop(0, nk, body, 0)

        scatter_copy(0).wait()
        scatter_copy(1).wait()
        _writeback(oacc, opart_hbm, cid, sid)

    return pass_b


def kernel(x, edge_index, batch, W1, a_src1, a_dst1, b1, lnw1, lnb1,
           W2, a_src2, a_dst2, b2, lnw2, lnb2):
    n = x.shape[0]
    e_in = edge_index.shape[1]
    e_tot = e_in + n
    q = 2 * KA
    epw = ((e_tot + NW * q - 1) // (NW * q)) * q
    e_pad = epw * NW

    loop = jnp.arange(n, dtype=jnp.int32)
    pad = jnp.zeros((e_pad - e_tot,), dtype=jnp.int32)
    src = jnp.concatenate([edge_index[0], loop, pad])
    dst = jnp.concatenate([edge_index[1], loop, pad])

    pass_a = _make_pass_a(e_pad, e_tot)
    pass_b = _make_pass_b(e_pad, e_tot)

    def layer(xin, W, a_src, a_dst, bias, lnw, lnb):
        h, ta, td = _project(xin, W.astype(jnp.bfloat16), a_src, a_dst)
        dparts = pass_a(src, dst, ta, td)
        trd = _rcomb(dparts, td)
        oparts = pass_b(src, dst, ta, trd, h)
        return _finalize(oparts, xin, bias, lnw, lnb)

    y1 = layer(x, W1, a_src1, a_dst1, b1, lnw1, lnb1)
    y2 = layer(y1, W2, a_src2, a_dst2, b2, lnw2, lnb2)
    return y2
